# baseline (speedup 1.0000x reference)
import functools

import jax
import jax.numpy as jnp
import numpy as np
from jax import lax
from jax.experimental import pallas as pl
from jax.experimental.pallas import tpu as pltpu
from jax.experimental.pallas import tpu_sc as plsc

N = 10000
E = 160000
D = 256
G = 64
H = 64
H2 = 2 * H
EPS = 1e-5

NC, NS, L = 2, 16, 16
NW = NC * NS
CH = 128
CH2 = CH // 2
CPW = 40
EPW = CPW * CH
EP = NW * EPW
EP2 = EP // 2
E2 = E // 2
NVC = E // CH
NPAD = 10240
RPT = NPAD // NS
BLK = 4096
NB = 10
NBLK = N // NB

_mesh = plsc.VectorSubcoreMesh(core_axis_name="c", subcore_axis_name="s")
_sc_params = pltpu.CompilerParams(use_tc_tiling_on_sc=False,
                                  needs_layout_passes=False)

_PM = np.empty((H,), np.int32)
for _g in (0, 1):
    for _i in range(L):
        _PM[32 * _g + 2 * _i] = 32 * _g + _i
        _PM[32 * _g + 2 * _i + 1] = 32 * _g + L + _i

_QM = np.empty((H2,), np.int32)
for _i in range(H):
    _QM[2 * _i] = _i
    _QM[2 * _i + 1] = H + _i


def _proj_body(x_ref, wa_ref, wb_ref, oa_ref, ob_ref):
    xb = x_ref[...]
    oa_ref[...] = jnp.dot(xb, wa_ref[...],
                          preferred_element_type=jnp.float32
                          ).astype(jnp.bfloat16)
    ob_ref[...] = jnp.dot(xb, wb_ref[...],
                          preferred_element_type=jnp.float32
                          ).astype(jnp.bfloat16)


def _project(x, wa, wb):
    return pl.pallas_call(
        _proj_body,
        grid=(NB,),
        in_specs=[
            pl.BlockSpec((NBLK, D), lambda i: (i, 0)),
            pl.BlockSpec((D, H), lambda i: (0, 0)),
            pl.BlockSpec((D, H), lambda i: (0, 0)),
        ],
        out_specs=[pl.BlockSpec((NBLK, H), lambda i: (i, 0)),
                   pl.BlockSpec((NBLK, H), lambda i: (i, 0))],
        out_shape=[jax.ShapeDtypeStruct((N, H), jnp.bfloat16),
                   jax.ShapeDtypeStruct((N, H), jnp.bfloat16)],
    )(x, wa, wb)


@functools.partial(
    pl.kernel,
    out_type=[
        jax.ShapeDtypeStruct((EP2, H2), jnp.bfloat16),
        jax.ShapeDtypeStruct((NW, 2, H), jnp.float32),
    ],
    mesh=_mesh,
    scratch_types=[
        pltpu.VMEM((EPW,), jnp.int32),
        pltpu.VMEM((EPW,), jnp.int32),
        pltpu.VMEM((CH, H), jnp.bfloat16),
        pltpu.VMEM((CH, H), jnp.bfloat16),
        pltpu.VMEM((CH, H), jnp.bfloat16),
        pltpu.VMEM((CH, H), jnp.bfloat16),
        pltpu.VMEM((CH2, H2), jnp.bfloat16),
        pltpu.VMEM((CH2, H2), jnp.bfloat16),
        pltpu.VMEM((2, H), jnp.float32),
        pltpu.SemaphoreType.DMA,
        pltpu.SemaphoreType.DMA,
        pltpu.SemaphoreType.DMA,
        pltpu.SemaphoreType.DMA,
    ],
    compiler_params=_sc_params,
)
def _edge_gather(tabd_hbm, tabs_hbm, idxd_hbm, idxs_hbm, z1_hbm, st_hbm,
                 idxd_v, idxs_v, rd0, rs0, rd1, rs1, zout0, zout1, acc_v,
                 rsem0, rsem1, wsem0, wsem1):
    cid = lax.axis_index("c")
    sid = lax.axis_index("s")
    wid = sid * NC + cid
    base = wid * EPW
    base2 = wid * (EPW // 2)
    nv = jnp.minimum(jnp.maximum(NVC - wid * CPW, 0), CPW)

    pltpu.sync_copy(idxd_hbm.at[pl.ds(base, EPW)], idxd_v)
    pltpu.sync_copy(idxs_hbm.at[pl.ds(base, EPW)], idxs_v)

    bufs = ((rd0, rs0, zout0, rsem0, wsem0),
            (rd1, rs1, zout1, rsem1, wsem1))

    def issue(c, bset):
        rd, rs, _, rsem, _ = bufs[bset]
        off = pl.multiple_of(c * CH, 8)
        pltpu.async_copy(tabd_hbm.at[idxd_v.at[pl.ds(off, CH)]], rd, rsem)
        pltpu.async_copy(tabs_hbm.at[idxs_v.at[pl.ds(off, CH)]], rs, rsem)

    def drain_read(bset):
        rd, rs, _, rsem, _ = bufs[bset]
        pltpu.make_async_copy(tabd_hbm.at[pl.ds(0, CH)], rd, rsem).wait()
        pltpu.make_async_copy(tabs_hbm.at[pl.ds(0, CH)], rs, rsem).wait()

    def drain_write(bset):
        _, _, zout, _, wsem = bufs[bset]
        pltpu.make_async_copy(zout, z1_hbm.at[pl.ds(0, CH2)], wsem).wait()

    def compute_store(c, bset, accs):
        rd, rs, zout, _, wsem = bufs[bset]

        def row_body(k, accs):
            zs = [None] * 8
            for e in (0, 1):
                for g in (0, 1):
                    d0, d1 = plsc.unpack(
                        rd[2 * k + e, pl.ds(2 * L * g, 2 * L)],
                        format=plsc.PackFormat.INTERLEAVED)
                    s0, s1 = plsc.unpack(
                        rs[2 * k + e, pl.ds(2 * L * g, 2 * L)],
                        format=plsc.PackFormat.INTERLEAVED)
                    zs[4 * e + 2 * g] = d0 + s0
                    zs[4 * e + 2 * g + 1] = d1 + s1
            out = []
            for cc in range(4):
                za = zs[cc]
                zb = zs[4 + cc]
                zout[k, pl.ds(2 * L * cc, 2 * L)] = plsc.pack(
                    za, zb, format=plsc.PackFormat.INTERLEAVED)
                out.append(accs[cc] + za + zb)
                out.append(accs[4 + cc] + za * za + zb * zb)
            return tuple(out[0::2]) + tuple(out[1::2])

        accs = lax.fori_loop(0, CH2, row_body, accs)
        pltpu.async_copy(zout, z1_hbm.at[pl.ds(base2 + c * CH2, CH2)],
                         wsem)
        return accs

    z16 = jnp.zeros((L,), jnp.float32)
    issue(0, 0)

    def pair_body(j, accs):
        c0 = 2 * j
        issue(c0 + 1, 1)
        drain_read(0)

        @pl.when(j >= 1)
        def _():
            drain_write(0)

        accs = compute_store(c0, 0, accs)

        @pl.when(c0 + 2 < nv)
        def _():
            issue(c0 + 2, 0)

        drain_read(1)

        @pl.when(j >= 1)
        def _():
            drain_write(1)

        accs = compute_store(c0 + 1, 1, accs)
        return accs

    accs = lax.fori_loop(0, nv // 2, pair_body, (z16,) * 8)
    drain_write(0)
    drain_write(1)
    for cc in range(4):
        acc_v[0, pl.ds(cc * L, L)] = accs[cc]
        acc_v[1, pl.ds(cc * L, L)] = accs[4 + cc]
    pltpu.sync_copy(acc_v, st_hbm.at[wid])


def _mlp_body(kstats, emit_next, interleaved_in, *refs):
    if emit_next:
        (st_ref, g_ref, bt_ref, gn_ref, btn_ref, z_ref, w_ref,
         zo_ref, so_ref, stn_ref, acc_ref) = refs
    else:
        (st_ref, g_ref, bt_ref, z_ref, w_ref,
         zo_ref, so_ref, acc_ref) = refs
    i = pl.program_id(0)
    st = jnp.sum(st_ref[...], axis=0)
    m = st[0] * (1.0 / E)
    v = st[1] * (1.0 / E) - m * m
    s = g_ref[...] * lax.rsqrt(v + EPS)
    t = bt_ref[...] - m * s
    if interleaved_in:
        sel = (lax.shift_right_logical(
            lax.broadcasted_iota(jnp.int32, (H, H2), 1), 1)
            == lax.broadcasted_iota(jnp.int32, (H, H2), 0))
        s2 = jnp.sum(jnp.where(sel, s[:, None], 0.0), axis=0)
        t2 = jnp.sum(jnp.where(sel, t[:, None], 0.0), axis=0)
    else:
        s2 = jnp.concatenate([s, s])
        t2 = jnp.concatenate([t, t])
    zin = z_ref[...].astype(jnp.float32)
    h = jnp.maximum(zin * s2[None, :] + t2[None, :], 0.0)
    z2 = jnp.dot(h, w_ref[...], preferred_element_type=jnp.float32)
    zo_ref[...] = z2.astype(zo_ref.dtype)
    rows = i * BLK + lax.broadcasted_iota(jnp.int32, (BLK, 1), 0)
    z2m = jnp.where(rows < E2, z2, 0.0)
    cs = jnp.sum(z2m, axis=0)
    cq = jnp.sum(z2m * z2m, axis=0)
    ps = jnp.stack([cs[:H] + cs[H:], cq[:H] + cq[H:]])

    @pl.when(i == 0)
    def _():
        acc_ref[...] = jnp.zeros((2, H), jnp.float32)

    acc_ref[...] += ps
    a = acc_ref[...]
    so_ref[...] = a
    if emit_next:
        m2 = a[0] * (1.0 / E)
        v2 = a[1] * (1.0 / E) - m2 * m2
        sn = gn_ref[...] * lax.rsqrt(v2 + EPS)
        tn = btn_ref[...] - m2 * sn
        stn_ref[...] = jnp.stack([sn, tn])


def _mlp_stage(stats, g, bt, z, wd, gn=None, btn=None,
               out_dtype=jnp.float32, interleaved_in=False):
    emit_next = gn is not None
    kstats = stats.shape[0]
    vec_spec = pl.BlockSpec((H,), lambda i: (0,))
    in_specs = [pl.BlockSpec((kstats, 2, H), lambda i: (0, 0, 0)),
                vec_spec, vec_spec]
    ops = [stats, g, bt]
    if emit_next:
        in_specs += [vec_spec, vec_spec]
        ops += [gn, btn]
    in_specs += [pl.BlockSpec((BLK, H2), lambda i: (i, 0)),
                 pl.BlockSpec((H2, H2), lambda i: (0, 0))]
    ops += [z, wd]
    out_specs = [pl.BlockSpec((BLK, H2), lambda i: (i, 0)),
                 pl.BlockSpec((2, H), lambda i: (0, 0))]
    out_shape = [jax.ShapeDtypeStruct((EP2, H2), out_dtype),
                 jax.ShapeDtypeStruct((2, H), jnp.float32)]
    if emit_next:
        out_specs.append(pl.BlockSpec((2, H), lambda i: (0, 0)))
        out_shape.append(jax.ShapeDtypeStruct((2, H), jnp.float32))
    return pl.pallas_call(
        functools.partial(_mlp_body, kstats, emit_next, interleaved_in),
        grid=(EP2 // BLK,),
        in_specs=in_specs,
        out_specs=out_specs,
        out_shape=out_shape,
        scratch_shapes=[pltpu.VMEM((2, H), jnp.float32)],
    )(*ops)


H80 = 80


@functools.partial(
    pl.kernel,
    out_type=jax.ShapeDtypeStruct((NC, NPAD, H80), jnp.float32),
    mesh=_mesh,
    scratch_types=[
        pltpu.VMEM((CPW, CH), jnp.int32),
        pltpu.VMEM((CH2, H2), jnp.float32),
        pltpu.VMEM((CH2, H2), jnp.float32),
        pltpu.VMEM((CH, H80), jnp.float32),
        pltpu.VMEM((CH, H80), jnp.float32),
        pltpu.VMEM((2, H), jnp.float32),
        pltpu.VMEM_SHARED((NPAD, H80), jnp.float32),
        pltpu.SemaphoreType.DMA,
        pltpu.SemaphoreType.DMA,
        pltpu.SemaphoreType.DMA,
        pltpu.SemaphoreType.DMA,
    ],
    compiler_params=_sc_params,
)
def _edge_scatter(z3_hbm, st_hbm, dsts_hbm, out_hbm,
                  idx_v, zbuf0, zbuf1, scat0, scat1, st_v, out_sh,
                  rsem0, rsem1, wsem0, wsem1):
    cid = lax.axis_index("c")
    sid = lax.axis_index("s")
    wid = sid * NC + cid
    base2 = wid * (EPW // 2)
    nv = jnp.minimum(jnp.maximum(NVC - wid * CPW, 0), CPW)

    z16 = jnp.zeros((L,), jnp.float32)
    one0 = jnp.where(lax.iota(jnp.int32, L) == 0, 1.0, 0.0)

    def zrow(k, _):
        for cc in range(5):
            scat0[k, pl.ds(cc * L, L)] = z16
        return 0

    lax.fori_loop(0, CH, zrow, 0)
    for r in range(RPT // CH):
        pltpu.async_copy(scat0, out_sh.at[pl.ds(sid * RPT + r * CH, CH)],
                         wsem0)
    for r in range(RPT // CH):
        pltpu.make_async_copy(scat0, out_sh.at[pl.ds(0, CH)],
                              wsem0).wait()

    def prow(k, _):
        scat0[k, pl.ds(H, L)] = one0
        scat1[k, pl.ds(H, L)] = one0
        return 0

    lax.fori_loop(0, CH, prow, 0)

    pltpu.sync_copy(st_hbm, st_v)
    pltpu.sync_copy(dsts_hbm.at[wid], idx_v)
    plsc.subcore_barrier()
    svec = [st_v[0, pl.ds(cc * L, L)] for cc in range(4)]
    tvec = [st_v[1, pl.ds(cc * L, L)] for cc in range(4)]

    bufs = ((zbuf0, scat0, rsem0, wsem0), (zbuf1, scat1, rsem1, wsem1))

    def issue(c, bset):
        zbuf, _, rsem, _ = bufs[bset]
        pltpu.async_copy(z3_hbm.at[pl.ds(base2 + c * CH2, CH2)], zbuf,
                         rsem)

    def drain_read(bset):
        zbuf, _, rsem, _ = bufs[bset]
        pltpu.make_async_copy(z3_hbm.at[pl.ds(0, CH2)], zbuf, rsem).wait()

    def drain_write(bset):
        _, scat, _, wsem = bufs[bset]
        pltpu.make_async_copy(scat, out_sh.at[pl.ds(0, CH)], wsem).wait()

    def transform_scatter(c, bset):
        zbuf, scat, _, wsem = bufs[bset]

        def row_body(k, _):
            for cc in range(4):
                za = zbuf[k, pl.ds(cc * L, L)]
                zb = zbuf[k, pl.ds(H + cc * L, L)]
                scat[2 * k, pl.ds(cc * L, L)] = jnp.maximum(
                    za * svec[cc] + tvec[cc], 0.0)
                scat[2 * k + 1, pl.ds(cc * L, L)] = jnp.maximum(
                    zb * svec[cc] + tvec[cc], 0.0)
            return 0

        lax.fori_loop(0, CH2, row_body, 0)
        pltpu.async_copy(scat, out_sh.at[idx_v.at[c]], wsem, add=True)

    issue(0, 0)

    def pair_body(j, _):
        c0 = 2 * j
        issue(c0 + 1, 1)
        drain_read(0)

        @pl.when(j >= 1)
        def _():
            drain_write(0)

        transform_scatter(c0, 0)

        @pl.when(c0 + 2 < nv)
        def _():
            issue(c0 + 2, 0)

        drain_read(1)

        @pl.when(j >= 1)
        def _():
            drain_write(1)

        transform_scatter(c0 + 1, 1)
        return 0

    lax.fori_loop(0, nv // 2, pair_body, 0)
    drain_write(0)
    drain_write(1)
    plsc.subcore_barrier()
    pltpu.sync_copy(out_sh.at[pl.ds(sid * RPT, RPT)],
                    out_hbm.at[cid, pl.ds(sid * RPT, RPT)])


def _poolx_body(x_ref, b_ref, px_out, cnt_out, px_ref, cnt_ref):
    i = pl.program_id(0)

    @pl.when(i == 0)
    def _():
        px_ref[...] = jnp.zeros_like(px_ref)
        cnt_ref[...] = jnp.zeros_like(cnt_ref)

    b = b_ref[...].reshape(1, NBLK)
    onehot = (lax.broadcasted_iota(jnp.int32, (G, NBLK), 0)
              == b).astype(jnp.float32)
    px_ref[...] += jnp.dot(onehot, x_ref[...],
                           preferred_element_type=jnp.float32)
    cnt_ref[...] += jnp.broadcast_to(
        jnp.sum(onehot, axis=1, keepdims=True), (G, H2))
    px_out[...] = px_ref[...]
    cnt_out[...] = cnt_ref[...]


def _poolx(x, batch3):
    return pl.pallas_call(
        _poolx_body,
        grid=(NB,),
        in_specs=[
            pl.BlockSpec((NBLK, D), lambda i: (i, 0)),
            pl.BlockSpec((1, 1, NBLK), lambda i: (i, 0, 0)),
        ],
        out_specs=[pl.BlockSpec((G, D), lambda i: (0, 0)),
                   pl.BlockSpec((G, H2), lambda i: (0, 0))],
        out_shape=[jax.ShapeDtypeStruct((G, D), jnp.float32),
                   jax.ShapeDtypeStruct((G, H2), jnp.float32)],
        scratch_shapes=[
            pltpu.VMEM((G, D), jnp.float32),
            pltpu.VMEM((G, H2), jnp.float32),
        ],
    )(x, batch3)


def _final_body(part_ref, px_ref2, cnt_ref2, b_ref, v1a_ref, v1x_ref,
                c1_ref, v2_ref, c2_ref, o_ref, pa_ref):
    i = pl.program_id(0)

    @pl.when(i == 0)
    def _():
        pa_ref[...] = jnp.zeros_like(pa_ref)

    p = part_ref[...]
    degs = p[0, :, H:H + 1] + p[1, :, H:H + 1]
    agg = (p[0, :, :H] + p[1, :, :H]) / jnp.maximum(degs, 1.0)
    b = b_ref[...].reshape(1, NBLK)
    onehot = (lax.broadcasted_iota(jnp.int32, (G, NBLK), 0)
              == b).astype(jnp.float32)
    pa_ref[...] += jnp.dot(onehot, agg, preferred_element_type=jnp.float32)

    inv = 1.0 / jnp.maximum(cnt_ref2[...][:, :1], 1.0)
    y = jnp.maximum(
        jnp.dot(pa_ref[...] * inv, v1a_ref[...],
                preferred_element_type=jnp.float32)
        + jnp.dot(px_ref2[...] * inv, v1x_ref[...],
                  preferred_element_type=jnp.float32)
        + c1_ref[...][None, :], 0.0)
    y2 = jnp.dot(y, v2_ref[...], preferred_element_type=jnp.float32) \
        + c2_ref[...][None, :]
    o_ref[...] = 1.0 / (1.0 + jnp.exp(-y2))


def _final(part, px, cnt, batch3, v1a, v1x, c1, v2, c2):
    return pl.pallas_call(
        _final_body,
        grid=(NB,),
        in_specs=[
            pl.BlockSpec((NC, NBLK, H80), lambda i: (0, i, 0)),
            pl.BlockSpec((G, D), lambda i: (0, 0)),
            pl.BlockSpec((G, H2), lambda i: (0, 0)),
            pl.BlockSpec((1, 1, NBLK), lambda i: (i, 0, 0)),
            pl.BlockSpec((H, H2), lambda i: (0, 0)),
            pl.BlockSpec((D, H2), lambda i: (0, 0)),
            pl.BlockSpec((H2,), lambda i: (0,)),
            pl.BlockSpec((H2, 1), lambda i: (0, 0)),
            pl.BlockSpec((1,), lambda i: (0,)),
        ],
        out_specs=pl.BlockSpec((G, 1), lambda i: (0, 0)),
        out_shape=jax.ShapeDtypeStruct((G, 1), jnp.float32),
        scratch_shapes=[
            pltpu.VMEM((G, H), jnp.float32),
        ],
    )(part, px, cnt, batch3, v1a, v1x, c1, v2, c2)


def _blockdiag(w):
    z = jnp.zeros((H, H), jnp.float32)
    return jnp.concatenate([
        jnp.concatenate([w, z], axis=1),
        jnp.concatenate([z, w], axis=1),
    ], axis=0)


def kernel(x, edge_index, batch, W1, b1, g1, bt1, W2, b2, g2, bt2,
           W3, b3, g3, bt3, V1, c1, V2, c2):
    src = edge_index[0]
    dst = edge_index[1]
    idx_d = jnp.pad(dst, (0, EP - E))
    idx_s = jnp.pad(src, (0, EP - E))
    dsts = jnp.pad(dst, (0, EP - E)).reshape(NW, CPW, CH)
    batch3 = batch.reshape(NB, 1, NBLK)

    pm = jnp.asarray(_PM)
    tabd, tabs = _project(x, (W1[:D] - W1[D:])[:, pm], W1[D:][:, pm])
    z1, st1 = _edge_gather(tabd, tabs, idx_d, idx_s)
    z2, st2 = _mlp_stage(st1, g1, bt1, z1, _blockdiag(W2)[jnp.asarray(_QM)],
                         out_dtype=jnp.bfloat16, interleaved_in=True)
    z3, _, st3 = _mlp_stage(st2.reshape(1, 2, H), g2, bt2, z2,
                            _blockdiag(W3), gn=g3, btn=bt3)
    part = _edge_scatter(z3, st3, dsts)
    px, cnt = _poolx(x, batch3)
    return _final(part, px, cnt, batch3, V1[:H], V1[H:], c1, V2, c2)


# speedup vs baseline: 1.3001x; 1.3001x over previous
import functools

import jax
import jax.numpy as jnp
import numpy as np
from jax import lax
from jax.experimental import pallas as pl
from jax.experimental.pallas import tpu as pltpu
from jax.experimental.pallas import tpu_sc as plsc

N = 10000
E = 160000
D = 256
G = 64
H = 64
H2 = 2 * H
EPS = 1e-5

NC, NS, L = 2, 16, 16
NW = NC * NS
CH = 128
CH2 = CH // 2
CPW = 40
EPW = CPW * CH
EP = NW * EPW
EP2 = EP // 2
E2 = E // 2
NVC = E // CH
NPAD = 10240
RPT = NPAD // NS
BLK = 4096
NB = 10
NBLK = N // NB

_mesh = plsc.VectorSubcoreMesh(core_axis_name="c", subcore_axis_name="s")
_sc_params = pltpu.CompilerParams(use_tc_tiling_on_sc=False,
                                  needs_layout_passes=False)

_PM = np.empty((H,), np.int32)
for _g in (0, 1):
    for _i in range(L):
        _PM[32 * _g + 2 * _i] = 32 * _g + _i
        _PM[32 * _g + 2 * _i + 1] = 32 * _g + L + _i

_QM = np.empty((H2,), np.int32)
for _i in range(H):
    _QM[2 * _i] = _i
    _QM[2 * _i + 1] = H + _i


def _proj_body(x_ref, wa_ref, wb_ref, oa_ref, ob_ref):
    xb = x_ref[...]
    oa_ref[...] = jnp.dot(xb, wa_ref[...],
                          preferred_element_type=jnp.float32
                          ).astype(jnp.bfloat16)
    ob_ref[...] = jnp.dot(xb, wb_ref[...],
                          preferred_element_type=jnp.float32
                          ).astype(jnp.bfloat16)


def _project(x, wa, wb):
    return pl.pallas_call(
        _proj_body,
        grid=(NB,),
        in_specs=[
            pl.BlockSpec((NBLK, D), lambda i: (i, 0)),
            pl.BlockSpec((D, H), lambda i: (0, 0)),
            pl.BlockSpec((D, H), lambda i: (0, 0)),
        ],
        out_specs=[pl.BlockSpec((NBLK, H), lambda i: (i, 0)),
                   pl.BlockSpec((NBLK, H), lambda i: (i, 0))],
        out_shape=[jax.ShapeDtypeStruct((N, H), jnp.bfloat16),
                   jax.ShapeDtypeStruct((N, H), jnp.bfloat16)],
    )(x, wa, wb)


@functools.partial(
    pl.kernel,
    out_type=[
        jax.ShapeDtypeStruct((EP2, H2), jnp.float32),
        jax.ShapeDtypeStruct((NW, 2, H), jnp.float32),
    ],
    mesh=_mesh,
    scratch_types=[
        pltpu.VMEM((EPW,), jnp.int32),
        pltpu.VMEM((EPW,), jnp.int32),
        pltpu.VMEM((CH, H), jnp.bfloat16),
        pltpu.VMEM((CH, H), jnp.bfloat16),
        pltpu.VMEM((CH, H), jnp.bfloat16),
        pltpu.VMEM((CH, H), jnp.bfloat16),
        pltpu.VMEM((CH2, H2), jnp.float32),
        pltpu.VMEM((CH2, H2), jnp.float32),
        pltpu.VMEM((2, H), jnp.float32),
        pltpu.SemaphoreType.DMA,
        pltpu.SemaphoreType.DMA,
        pltpu.SemaphoreType.DMA,
        pltpu.SemaphoreType.DMA,
    ],
    compiler_params=_sc_params,
)
def _edge_gather(tabd_hbm, tabs_hbm, idxd_hbm, idxs_hbm, z1_hbm, st_hbm,
                 idxd_v, idxs_v, rd0, rs0, rd1, rs1, zout0, zout1, acc_v,
                 rsem0, rsem1, wsem0, wsem1):
    cid = lax.axis_index("c")
    sid = lax.axis_index("s")
    wid = sid * NC + cid
    base = wid * EPW
    base2 = wid * (EPW // 2)
    nv = jnp.minimum(jnp.maximum(NVC - wid * CPW, 0), CPW)

    pltpu.sync_copy(idxd_hbm.at[pl.ds(base, EPW)], idxd_v)
    pltpu.sync_copy(idxs_hbm.at[pl.ds(base, EPW)], idxs_v)

    bufs = ((rd0, rs0, zout0, rsem0, wsem0),
            (rd1, rs1, zout1, rsem1, wsem1))

    def issue(c, bset):
        rd, rs, _, rsem, _ = bufs[bset]
        off = pl.multiple_of(c * CH, 8)
        pltpu.async_copy(tabd_hbm.at[idxd_v.at[pl.ds(off, CH)]], rd, rsem)
        pltpu.async_copy(tabs_hbm.at[idxs_v.at[pl.ds(off, CH)]], rs, rsem)

    def drain_read(bset):
        rd, rs, _, rsem, _ = bufs[bset]
        pltpu.make_async_copy(tabd_hbm.at[pl.ds(0, CH)], rd, rsem).wait()
        pltpu.make_async_copy(tabs_hbm.at[pl.ds(0, CH)], rs, rsem).wait()

    def drain_write(bset):
        _, _, zout, _, wsem = bufs[bset]
        pltpu.make_async_copy(zout, z1_hbm.at[pl.ds(0, CH2)], wsem).wait()

    def compute_store(c, bset, accs):
        rd, rs, zout, _, wsem = bufs[bset]

        def row_body(k, accs):
            zs = [None] * 8
            for e in (0, 1):
                for g in (0, 1):
                    d0, d1 = plsc.unpack(
                        rd[2 * k + e, pl.ds(2 * L * g, 2 * L)],
                        format=plsc.PackFormat.INTERLEAVED)
                    s0, s1 = plsc.unpack(
                        rs[2 * k + e, pl.ds(2 * L * g, 2 * L)],
                        format=plsc.PackFormat.INTERLEAVED)
                    zs[4 * e + 2 * g] = d0 + s0
                    zs[4 * e + 2 * g + 1] = d1 + s1
            out = []
            for cc in range(4):
                za = zs[cc]
                zb = zs[4 + cc]
                zout[k, pl.ds(cc * L, L)] = za
                zout[k, pl.ds(H + cc * L, L)] = zb
                out.append(accs[cc] + za + zb)
                out.append(accs[4 + cc] + za * za + zb * zb)
            return tuple(out[0::2]) + tuple(out[1::2])

        accs = lax.fori_loop(0, CH2, row_body, accs)
        pltpu.async_copy(zout, z1_hbm.at[pl.ds(base2 + c * CH2, CH2)],
                         wsem)
        return accs

    z16 = jnp.zeros((L,), jnp.float32)
    issue(0, 0)

    def pair_body(j, accs):
        c0 = 2 * j
        issue(c0 + 1, 1)
        drain_read(0)

        @pl.when(j >= 1)
        def _():
            drain_write(0)

        accs = compute_store(c0, 0, accs)

        @pl.when(c0 + 2 < nv)
        def _():
            issue(c0 + 2, 0)

        drain_read(1)

        @pl.when(j >= 1)
        def _():
            drain_write(1)

        accs = compute_store(c0 + 1, 1, accs)
        return accs

    accs = lax.fori_loop(0, nv // 2, pair_body, (z16,) * 8)
    drain_write(0)
    drain_write(1)
    for cc in range(4):
        acc_v[0, pl.ds(cc * L, L)] = accs[cc]
        acc_v[1, pl.ds(cc * L, L)] = accs[4 + cc]
    pltpu.sync_copy(acc_v, st_hbm.at[wid])


def _mlp_body(kstats, emit_next, interleaved_in, *refs):
    if emit_next:
        (st_ref, g_ref, bt_ref, gn_ref, btn_ref, z_ref, w_ref,
         zo_ref, so_ref, stn_ref, acc_ref) = refs
    else:
        (st_ref, g_ref, bt_ref, z_ref, w_ref,
         zo_ref, so_ref, acc_ref) = refs
    i = pl.program_id(0)
    st = jnp.sum(st_ref[...], axis=0)
    m = st[0] * (1.0 / E)
    v = st[1] * (1.0 / E) - m * m
    s = g_ref[...] * lax.rsqrt(v + EPS)
    t = bt_ref[...] - m * s
    if interleaved_in:
        sel = (lax.shift_right_logical(
            lax.broadcasted_iota(jnp.int32, (H, H2), 1), 1)
            == lax.broadcasted_iota(jnp.int32, (H, H2), 0))
        s2 = jnp.sum(jnp.where(sel, s[:, None], 0.0), axis=0)
        t2 = jnp.sum(jnp.where(sel, t[:, None], 0.0), axis=0)
    else:
        s2 = jnp.concatenate([s, s])
        t2 = jnp.concatenate([t, t])
    zin = z_ref[...].astype(jnp.float32)
    h = jnp.maximum(zin * s2[None, :] + t2[None, :], 0.0)
    z2 = jnp.dot(h, w_ref[...], preferred_element_type=jnp.float32)
    zo_ref[...] = z2.astype(zo_ref.dtype)
    rows = i * BLK + lax.broadcasted_iota(jnp.int32, (BLK, 1), 0)
    z2m = jnp.where(rows < E2, z2, 0.0)
    cs = jnp.sum(z2m, axis=0)
    cq = jnp.sum(z2m * z2m, axis=0)
    ps = jnp.stack([cs[:H] + cs[H:], cq[:H] + cq[H:]])

    @pl.when(i == 0)
    def _():
        acc_ref[...] = jnp.zeros((2, H), jnp.float32)

    acc_ref[...] += ps
    a = acc_ref[...]
    so_ref[...] = a
    if emit_next:
        m2 = a[0] * (1.0 / E)
        v2 = a[1] * (1.0 / E) - m2 * m2
        sn = gn_ref[...] * lax.rsqrt(v2 + EPS)
        tn = btn_ref[...] - m2 * sn
        stn_ref[...] = jnp.stack([sn, tn])


def _mlp_stage(stats, g, bt, z, wd, gn=None, btn=None,
               out_dtype=jnp.float32, interleaved_in=False):
    emit_next = gn is not None
    kstats = stats.shape[0]
    vec_spec = pl.BlockSpec((H,), lambda i: (0,))
    in_specs = [pl.BlockSpec((kstats, 2, H), lambda i: (0, 0, 0)),
                vec_spec, vec_spec]
    ops = [stats, g, bt]
    if emit_next:
        in_specs += [vec_spec, vec_spec]
        ops += [gn, btn]
    in_specs += [pl.BlockSpec((BLK, H2), lambda i: (i, 0)),
                 pl.BlockSpec((H2, H2), lambda i: (0, 0))]
    ops += [z, wd]
    out_specs = [pl.BlockSpec((BLK, H2), lambda i: (i, 0)),
                 pl.BlockSpec((2, H), lambda i: (0, 0))]
    out_shape = [jax.ShapeDtypeStruct((EP2, H2), out_dtype),
                 jax.ShapeDtypeStruct((2, H), jnp.float32)]
    if emit_next:
        out_specs.append(pl.BlockSpec((2, H), lambda i: (0, 0)))
        out_shape.append(jax.ShapeDtypeStruct((2, H), jnp.float32))
    return pl.pallas_call(
        functools.partial(_mlp_body, kstats, emit_next, interleaved_in),
        grid=(EP2 // BLK,),
        in_specs=in_specs,
        out_specs=out_specs,
        out_shape=out_shape,
        scratch_shapes=[pltpu.VMEM((2, H), jnp.float32)],
    )(*ops)


H80 = 80


@functools.partial(
    pl.kernel,
    out_type=jax.ShapeDtypeStruct((NC, NPAD, H80), jnp.float32),
    mesh=_mesh,
    scratch_types=[
        pltpu.VMEM((CPW, CH), jnp.int32),
        pltpu.VMEM((CH2, H2), jnp.float32),
        pltpu.VMEM((CH2, H2), jnp.float32),
        pltpu.VMEM((CH, H80), jnp.float32),
        pltpu.VMEM((CH, H80), jnp.float32),
        pltpu.VMEM((2, H), jnp.float32),
        pltpu.VMEM_SHARED((NPAD, H80), jnp.float32),
        pltpu.SemaphoreType.DMA,
        pltpu.SemaphoreType.DMA,
        pltpu.SemaphoreType.DMA,
        pltpu.SemaphoreType.DMA,
    ],
    compiler_params=_sc_params,
)
def _edge_scatter(z3_hbm, st_hbm, dsts_hbm, out_hbm,
                  idx_v, zbuf0, zbuf1, scat0, scat1, st_v, out_sh,
                  rsem0, rsem1, wsem0, wsem1):
    cid = lax.axis_index("c")
    sid = lax.axis_index("s")
    wid = sid * NC + cid
    base2 = wid * (EPW // 2)
    nv = jnp.minimum(jnp.maximum(NVC - wid * CPW, 0), CPW)

    z16 = jnp.zeros((L,), jnp.float32)
    one0 = jnp.where(lax.iota(jnp.int32, L) == 0, 1.0, 0.0)

    def zrow(k, _):
        for cc in range(5):
            scat0[k, pl.ds(cc * L, L)] = z16
        return 0

    lax.fori_loop(0, CH, zrow, 0)
    for r in range(RPT // CH):
        pltpu.async_copy(scat0, out_sh.at[pl.ds(sid * RPT + r * CH, CH)],
                         wsem0)
    for r in range(RPT // CH):
        pltpu.make_async_copy(scat0, out_sh.at[pl.ds(0, CH)],
                              wsem0).wait()

    def prow(k, _):
        scat0[k, pl.ds(H, L)] = one0
        scat1[k, pl.ds(H, L)] = one0
        return 0

    lax.fori_loop(0, CH, prow, 0)

    pltpu.sync_copy(st_hbm, st_v)
    pltpu.sync_copy(dsts_hbm.at[wid], idx_v)
    plsc.subcore_barrier()
    svec = [st_v[0, pl.ds(cc * L, L)] for cc in range(4)]
    tvec = [st_v[1, pl.ds(cc * L, L)] for cc in range(4)]

    bufs = ((zbuf0, scat0, rsem0, wsem0), (zbuf1, scat1, rsem1, wsem1))

    def issue(c, bset):
        zbuf, _, rsem, _ = bufs[bset]
        pltpu.async_copy(z3_hbm.at[pl.ds(base2 + c * CH2, CH2)], zbuf,
                         rsem)

    def drain_read(bset):
        zbuf, _, rsem, _ = bufs[bset]
        pltpu.make_async_copy(z3_hbm.at[pl.ds(0, CH2)], zbuf, rsem).wait()

    def drain_write(bset):
        _, scat, _, wsem = bufs[bset]
        pltpu.make_async_copy(scat, out_sh.at[pl.ds(0, CH)], wsem).wait()

    def transform_scatter(c, bset):
        zbuf, scat, _, wsem = bufs[bset]

        def row_body(k, _):
            for cc in range(4):
                za = zbuf[k, pl.ds(cc * L, L)]
                zb = zbuf[k, pl.ds(H + cc * L, L)]
                scat[2 * k, pl.ds(cc * L, L)] = jnp.maximum(
                    za * svec[cc] + tvec[cc], 0.0)
                scat[2 * k + 1, pl.ds(cc * L, L)] = jnp.maximum(
                    zb * svec[cc] + tvec[cc], 0.0)
            return 0

        lax.fori_loop(0, CH2, row_body, 0)
        pltpu.async_copy(scat, out_sh.at[idx_v.at[c]], wsem, add=True)

    issue(0, 0)

    def pair_body(j, _):
        c0 = 2 * j
        issue(c0 + 1, 1)
        drain_read(0)

        @pl.when(j >= 1)
        def _():
            drain_write(0)

        transform_scatter(c0, 0)

        @pl.when(c0 + 2 < nv)
        def _():
            issue(c0 + 2, 0)

        drain_read(1)

        @pl.when(j >= 1)
        def _():
            drain_write(1)

        transform_scatter(c0 + 1, 1)
        return 0

    lax.fori_loop(0, nv // 2, pair_body, 0)
    drain_write(0)
    drain_write(1)
    plsc.subcore_barrier()
    pltpu.sync_copy(out_sh.at[pl.ds(sid * RPT, RPT)],
                    out_hbm.at[cid, pl.ds(sid * RPT, RPT)])


def _poolx_body(x_ref, b_ref, px_out, cnt_out, px_ref, cnt_ref):
    i = pl.program_id(0)

    @pl.when(i == 0)
    def _():
        px_ref[...] = jnp.zeros_like(px_ref)
        cnt_ref[...] = jnp.zeros_like(cnt_ref)

    b = b_ref[...].reshape(1, NBLK)
    onehot = (lax.broadcasted_iota(jnp.int32, (G, NBLK), 0)
              == b).astype(jnp.float32)
    px_ref[...] += jnp.dot(onehot, x_ref[...],
                           preferred_element_type=jnp.float32)
    cnt_ref[...] += jnp.broadcast_to(
        jnp.sum(onehot, axis=1, keepdims=True), (G, H2))
    px_out[...] = px_ref[...]
    cnt_out[...] = cnt_ref[...]


def _poolx(x, batch3):
    return pl.pallas_call(
        _poolx_body,
        grid=(NB,),
        in_specs=[
            pl.BlockSpec((NBLK, D), lambda i: (i, 0)),
            pl.BlockSpec((1, 1, NBLK), lambda i: (i, 0, 0)),
        ],
        out_specs=[pl.BlockSpec((G, D), lambda i: (0, 0)),
                   pl.BlockSpec((G, H2), lambda i: (0, 0))],
        out_shape=[jax.ShapeDtypeStruct((G, D), jnp.float32),
                   jax.ShapeDtypeStruct((G, H2), jnp.float32)],
        scratch_shapes=[
            pltpu.VMEM((G, D), jnp.float32),
            pltpu.VMEM((G, H2), jnp.float32),
        ],
    )(x, batch3)


def _final_body(part_ref, px_ref2, cnt_ref2, b_ref, v1a_ref, v1x_ref,
                c1_ref, v2_ref, c2_ref, o_ref, pa_ref):
    i = pl.program_id(0)

    @pl.when(i == 0)
    def _():
        pa_ref[...] = jnp.zeros_like(pa_ref)

    p = part_ref[...]
    degs = p[0, :, H:H + 1] + p[1, :, H:H + 1]
    agg = (p[0, :, :H] + p[1, :, :H]) / jnp.maximum(degs, 1.0)
    b = b_ref[...].reshape(1, NBLK)
    onehot = (lax.broadcasted_iota(jnp.int32, (G, NBLK), 0)
              == b).astype(jnp.float32)
    pa_ref[...] += jnp.dot(onehot, agg, preferred_element_type=jnp.float32)

    inv = 1.0 / jnp.maximum(cnt_ref2[...][:, :1], 1.0)
    y = jnp.maximum(
        jnp.dot(pa_ref[...] * inv, v1a_ref[...],
                preferred_element_type=jnp.float32)
        + jnp.dot(px_ref2[...] * inv, v1x_ref[...],
                  preferred_element_type=jnp.float32)
        + c1_ref[...][None, :], 0.0)
    y2 = jnp.dot(y, v2_ref[...], preferred_element_type=jnp.float32) \
        + c2_ref[...][None, :]
    o_ref[...] = 1.0 / (1.0 + jnp.exp(-y2))


def _final(part, px, cnt, batch3, v1a, v1x, c1, v2, c2):
    return pl.pallas_call(
        _final_body,
        grid=(NB,),
        in_specs=[
            pl.BlockSpec((NC, NBLK, H80), lambda i: (0, i, 0)),
            pl.BlockSpec((G, D), lambda i: (0, 0)),
            pl.BlockSpec((G, H2), lambda i: (0, 0)),
            pl.BlockSpec((1, 1, NBLK), lambda i: (i, 0, 0)),
            pl.BlockSpec((H, H2), lambda i: (0, 0)),
            pl.BlockSpec((D, H2), lambda i: (0, 0)),
            pl.BlockSpec((H2,), lambda i: (0,)),
            pl.BlockSpec((H2, 1), lambda i: (0, 0)),
            pl.BlockSpec((1,), lambda i: (0,)),
        ],
        out_specs=pl.BlockSpec((G, 1), lambda i: (0, 0)),
        out_shape=jax.ShapeDtypeStruct((G, 1), jnp.float32),
        scratch_shapes=[
            pltpu.VMEM((G, H), jnp.float32),
        ],
    )(part, px, cnt, batch3, v1a, v1x, c1, v2, c2)


def _blockdiag(w):
    z = jnp.zeros((H, H), jnp.float32)
    return jnp.concatenate([
        jnp.concatenate([w, z], axis=1),
        jnp.concatenate([z, w], axis=1),
    ], axis=0)


def kernel(x, edge_index, batch, W1, b1, g1, bt1, W2, b2, g2, bt2,
           W3, b3, g3, bt3, V1, c1, V2, c2):
    src = edge_index[0]
    dst = edge_index[1]
    idx_d = jnp.pad(dst, (0, EP - E))
    idx_s = jnp.pad(src, (0, EP - E))
    dsts = jnp.pad(dst, (0, EP - E)).reshape(NW, CPW, CH)
    batch3 = batch.reshape(NB, 1, NBLK)

    pm = jnp.asarray(_PM)
    tabd, tabs = _project(x, (W1[:D] - W1[D:])[:, pm], W1[D:][:, pm])
    z1, st1 = _edge_gather(tabd, tabs, idx_d, idx_s)
    z2, st2 = _mlp_stage(st1, g1, bt1, z1, _blockdiag(W2),
                         out_dtype=jnp.bfloat16)
    z3, _, st3 = _mlp_stage(st2.reshape(1, 2, H), g2, bt2, z2,
                            _blockdiag(W3), gn=g3, btn=bt3)
    part = _edge_scatter(z3, st3, dsts)
    px, cnt = _poolx(x, batch3)
    return _final(part, px, cnt, batch3, V1[:H], V1[H:], c1, V2, c2)
